# baseline (speedup 1.0000x reference)
import functools

import jax
import jax.numpy as jnp
from jax.experimental import pallas as pl
from jax.experimental.pallas import tpu as pltpu

_MiB = 1024 * 1024


def _probe_kernel(x_ref, w1t_ref, w2t_ref, o_ref, *, inv_hw):
    x = x_ref[...]
    pooled = jnp.sum(x, axis=2, dtype=jnp.float32) * inv_hw
    hidden = jnp.maximum(
        jnp.dot(pooled, w1t_ref[...], preferred_element_type=jnp.float32), 0.0)
    s = jax.nn.sigmoid(
        jnp.dot(hidden, w2t_ref[...], preferred_element_type=jnp.float32))
    B = x.shape[0]
    o_ref[...] = jnp.broadcast_to(s[0, 0], o_ref.shape)


def kernel(x, w1, w2):
    N, C, H, W = x.shape
    HW = H * W
    Cr = w1.shape[0]
    x_flat = x.reshape(N, C, HW)
    w1t = w1.astype(jnp.float32).T
    w2t = w2.astype(jnp.float32).T
    B = 8
    out = pl.pallas_call(
        functools.partial(_probe_kernel, inv_hw=1.0 / HW),
        out_shape=jax.ShapeDtypeStruct((N, 16, 6272), jnp.float32),
        grid=(N // B,),
        in_specs=[
            pl.BlockSpec((B, C, HW), lambda n: (n, 0, 0)),
            pl.BlockSpec((C, Cr), lambda n: (0, 0)),
            pl.BlockSpec((Cr, C), lambda n: (0, 0)),
        ],
        out_specs=pl.BlockSpec((B, 16, 6272), lambda n: (n, 0, 0)),
        compiler_params=pltpu.CompilerParams(
            dimension_semantics=("parallel",),
            vmem_limit_bytes=48 * _MiB,
        ),
    )(x_flat, w1t, w2t)
    return out.reshape(N, C, H, W)


# speedup vs baseline: 1.5243x; 1.5243x over previous
import functools

import jax
import jax.numpy as jnp
from jax.experimental import pallas as pl
from jax.experimental.pallas import tpu as pltpu

_MiB = 1024 * 1024


def _se_manual_kernel(x_hbm, w1t_ref, w2t_ref, o_hbm,
                      x_buf, o_buf, in_sems, out_sems,
                      *, B, S, NB, SP, inv_hw):
    B2 = B // SP

    H2 = S * B2

    def start_in(step, slot):
        for j in range(SP):
            pltpu.make_async_copy(
                x_hbm.at[pl.ds(step * B2 + j * H2, B2)],
                x_buf.at[slot, pl.ds(j * B2, B2)],
                in_sems.at[slot, j]).start()

    def wait_in(slot):
        for j in range(SP):
            pltpu.make_async_copy(
                x_hbm.at[pl.ds(0, B2)],
                x_buf.at[slot, pl.ds(j * B2, B2)],
                in_sems.at[slot, j]).wait()

    def start_out(step, slot):
        for j in range(SP):
            pltpu.make_async_copy(
                o_buf.at[slot, pl.ds(j * B2, B2)],
                o_hbm.at[pl.ds(step * B2 + j * H2, B2)],
                out_sems.at[slot, j]).start()

    def wait_out(slot):
        for j in range(SP):
            pltpu.make_async_copy(
                o_buf.at[slot, pl.ds(j * B2, B2)],
                o_hbm.at[pl.ds(0, B2)],
                out_sems.at[slot, j]).wait()

    D = NB - 1
    for p in range(min(D, S)):
        start_in(p, p % NB)

    w1t = w1t_ref[...]
    w2t = w2t_ref[...]

    def body(i, _):
        slot = jax.lax.rem(i, NB)

        @pl.when(i >= NB)
        def _():
            wait_out(slot)

        wait_in(slot)
        x = x_buf[slot]
        pooled = jnp.sum(x, axis=2, dtype=jnp.float32) * inv_hw
        hidden = jnp.maximum(
            jnp.dot(pooled, w1t, preferred_element_type=jnp.float32), 0.0)
        s = jax.nn.sigmoid(
            jnp.dot(hidden, w2t, preferred_element_type=jnp.float32))
        o_buf[slot] = x * s[:, :, None]
        start_out(i, slot)

        @pl.when(i + D < S)
        def _():
            start_in(i + D, jax.lax.rem(i + D, NB))
        return 0

    jax.lax.fori_loop(0, S, body, 0)
    for q in range(min(NB, S)):
        wait_out((S - 1 - q) % NB)


def kernel(x, w1, w2):
    N, C, H, W = x.shape
    HW = H * W
    Cr = w1.shape[0]
    x_flat = x.reshape(N, C, HW)
    w1t = w1.astype(jnp.float32).T
    w2t = w2.astype(jnp.float32).T

    B = 8
    while N % B:
        B //= 2
    S = N // B
    NB = min(4, S)
    SP = 2 if B % 2 == 0 else 1
    f32 = jnp.float32

    out_flat = pl.pallas_call(
        functools.partial(_se_manual_kernel, B=B, S=S, NB=NB, SP=SP,
                          inv_hw=1.0 / HW),
        out_shape=jax.ShapeDtypeStruct((N, C, HW), x.dtype),
        in_specs=[
            pl.BlockSpec(memory_space=pl.ANY),
            pl.BlockSpec((C, Cr), lambda: (0, 0)),
            pl.BlockSpec((Cr, C), lambda: (0, 0)),
        ],
        out_specs=pl.BlockSpec(memory_space=pl.ANY),
        scratch_shapes=[
            pltpu.VMEM((NB, B, C, HW), f32),
            pltpu.VMEM((NB, B, C, HW), f32),
            pltpu.SemaphoreType.DMA((NB, SP)),
            pltpu.SemaphoreType.DMA((NB, SP)),
        ],
        compiler_params=pltpu.CompilerParams(
            vmem_limit_bytes=56 * _MiB,
        ),
    )(x_flat, w1t, w2t)
    return out_flat.reshape(N, C, H, W)
